# baseline (speedup 1.0000x reference)
import jax
import jax.numpy as jnp
from jax.experimental import pallas as pl


TN = 512


def _chamfer_kernel(x1_ref, x2_ref, d1_ref, d2_ref):
    b = pl.program_id(0)
    i = pl.program_id(1)
    a = x1_ref[0]
    c = x2_ref[0]
    a_sq = jnp.sum(a * a, axis=1)
    c_sq = jnp.sum(c * c, axis=1)
    cross = jax.lax.dot_general(
        a, c, (((1,), (1,)), ((), ())),
        preferred_element_type=jnp.float32)
    d = a_sq[:, None] + c_sq[None, :] - 2.0 * cross
    d = jnp.maximum(d, 0.0)
    d1_ref[pl.ds(b, 1), pl.ds(i * TN, TN)] = jnp.min(d, axis=0 + 1)[None, :]
    part2 = jnp.min(d, axis=0)[None, :]

    @pl.when(i == 0)
    def _():
        d2_ref[pl.ds(b, 1), :] = part2

    @pl.when(i != 0)
    def _():
        d2_ref[pl.ds(b, 1), :] = jnp.minimum(d2_ref[pl.ds(b, 1), :], part2)


@jax.jit
def kernel(xyz1, xyz2):
    B, N, _ = xyz1.shape
    M = xyz2.shape[1]
    grid = (B, N // TN)
    d1, d2 = pl.pallas_call(
        _chamfer_kernel,
        grid=grid,
        in_specs=[
            pl.BlockSpec((1, TN, 3), lambda b, i: (b, i, 0)),
            pl.BlockSpec((1, M, 3), lambda b, i: (b, 0, 0)),
        ],
        out_specs=[
            pl.BlockSpec((B, N), lambda b, i: (0, 0)),
            pl.BlockSpec((B, M), lambda b, i: (0, 0)),
        ],
        out_shape=[
            jax.ShapeDtypeStruct((B, N), jnp.float32),
            jax.ShapeDtypeStruct((B, M), jnp.float32),
        ],
    )(xyz1, xyz2)
    return (d1, d2)
